# baseline (speedup 1.0000x reference)
import functools

import jax
import jax.numpy as jnp
from jax import lax
from jax.experimental import pallas as pl
from jax.experimental.pallas import tpu as pltpu
from jax.experimental.pallas import tpu_sc as plsc

N = 10000
E = 320000
D = 128
H = 128
NCELLS = 5000
LABSM = 0.1

NCORE = 2
NSUB = 16
NW = NCORE * NSUB
EPW = E // NW
CH = 80
NSTAGE = 25
NGROUP = EPW // (CH * NSTAGE)
NCHUNK = EPW // CH
NP = 10240
ROWS_PER_SUB = NP // NSUB
ZR = 32


HCH = CH // 2


def _gather2(x_hbm, src_c, k, rows, sem):
    pltpu.async_copy(x_hbm.at[src_c.at[k, pl.ds(0, HCH)]],
                     rows.at[pl.ds(0, HCH)], sem)
    pltpu.async_copy(x_hbm.at[src_c.at[k, pl.ds(HCH, HCH)]],
                     rows.at[pl.ds(HCH, HCH)], sem)


def _sc_body(x_hbm, src_hbm, dst_hbm, agg_out, deg_out,
             src_c, dst_c, rows0, rows1, ones_v, zb_v, zbd_v,
             agg_sp, deg_sp, sem_g, sem_s, sem_o):
    c = lax.axis_index("c")
    s = lax.axis_index("s")
    wid = s * NCORE + c

    ones16 = jnp.full((16,), 1.0, jnp.float32)
    zero16 = jnp.zeros((16,), jnp.float32)

    def fill_ones(i, carry):
        ones_v[i, :] = ones16
        return carry

    lax.fori_loop(0, CH, fill_ones, 0)

    def fill_zb(k, carry):
        zb_v[k // 8, pl.ds((k % 8) * 16, 16)] = zero16
        return carry

    lax.fori_loop(0, ZR * 8, fill_zb, 0)

    def fill_zbd(i, carry):
        zbd_v[i, :] = zero16
        return carry

    lax.fori_loop(0, ZR, fill_zbd, 0)

    def zero_slice(t, carry):
        base = s * ROWS_PER_SUB + t * ZR
        pltpu.sync_copy(zb_v, agg_sp.at[pl.ds(base, ZR)])
        pltpu.sync_copy(zbd_v, deg_sp.at[pl.ds(base, ZR)])
        return carry

    lax.fori_loop(0, ROWS_PER_SUB // ZR, zero_slice, 0)
    plsc.subcore_barrier()

    gwait = x_hbm.at[pl.ds(0, CH)]

    def group(t, carry):
        pltpu.sync_copy(src_hbm.at[wid, t], src_c)
        pltpu.sync_copy(dst_hbm.at[wid, t], dst_c)
        _gather2(x_hbm, src_c, 0, rows0, sem_g)

        def chunk(k, carry2):
            even = k % 2 == 0
            pltpu.make_async_copy(gwait, rows0, sem_g).wait()

            @pl.when(k > 0)
            def _():
                pltpu.make_async_copy(rows0, gwait, sem_s).wait()

            @pl.when(k + 1 < NSTAGE)
            def _():
                @pl.when(even)
                def _():
                    _gather2(x_hbm, src_c, k + 1, rows1, sem_g)

                @pl.when(jnp.logical_not(even))
                def _():
                    _gather2(x_hbm, src_c, k + 1, rows0, sem_g)

            @pl.when(even)
            def _():
                pltpu.async_copy(rows0, agg_sp.at[dst_c.at[k]], sem_s,
                                 add=True)

            @pl.when(jnp.logical_not(even))
            def _():
                pltpu.async_copy(rows1, agg_sp.at[dst_c.at[k]], sem_s,
                                 add=True)

            pltpu.async_copy(ones_v, deg_sp.at[dst_c.at[k]], sem_o, add=True)
            return carry2

        lax.fori_loop(0, NSTAGE, chunk, 0)
        pltpu.make_async_copy(rows0, gwait, sem_s).wait()

        def drain(k, carry2):
            pltpu.make_async_copy(ones_v, deg_sp.at[pl.ds(0, CH)],
                                  sem_o).wait()
            return carry2

        lax.fori_loop(0, NSTAGE, drain, 0)
        return carry

    lax.fori_loop(0, NGROUP, group, 0)
    plsc.subcore_barrier()

    base = s * ROWS_PER_SUB
    pltpu.sync_copy(agg_sp.at[pl.ds(base, ROWS_PER_SUB)],
                    agg_out.at[pl.ds(c * NP + base, ROWS_PER_SUB)])
    pltpu.sync_copy(deg_sp.at[pl.ds(base, ROWS_PER_SUB)],
                    deg_out.at[pl.ds(c * NP + base, ROWS_PER_SUB)])


@functools.cache
def _sc_segment_sum():
    return pl.kernel(
        _sc_body,
        out_type=(
            jax.ShapeDtypeStruct((NCORE * NP, D), jnp.float32),
            jax.ShapeDtypeStruct((NCORE * NP, 16), jnp.float32),
        ),
        mesh=plsc.VectorSubcoreMesh(core_axis_name="c", subcore_axis_name="s"),
        compiler_params=pltpu.CompilerParams(use_tc_tiling_on_sc=False),
        scratch_types=[
            pltpu.VMEM((NSTAGE, CH), jnp.int32),
            pltpu.VMEM((NSTAGE, CH), jnp.int32),
            pltpu.VMEM((CH, D), jnp.float32),
            pltpu.VMEM((CH, D), jnp.float32),
            pltpu.VMEM((CH, 16), jnp.float32),
            pltpu.VMEM((ZR, D), jnp.float32),
            pltpu.VMEM((ZR, 16), jnp.float32),
            pltpu.VMEM_SHARED((NP, D), jnp.float32),
            pltpu.VMEM_SHARED((NP, 16), jnp.float32),
            pltpu.SemaphoreType.DMA,
            pltpu.SemaphoreType.DMA,
            pltpu.SemaphoreType.DMA,
        ],
    )


BR_B = 200
NBLK_B = NCELLS // BR_B


def _emb(xb, aggp, degp, w1t, w1b, b1):
    agg = aggp[0] + aggp[1]
    deg = degp[0, :, 0:1] + degp[1, :, 0:1]
    aggm = agg / jnp.maximum(deg, 1.0)
    h = (jnp.dot(xb, w1t, preferred_element_type=jnp.float32)
         + jnp.dot(aggm, w1b, preferred_element_type=jnp.float32)
         + b1)
    return jnp.maximum(h, 0.0)


def _tc_b2_body(xg_ref, aggg_ref, degg_ref, xc_ref, aggc_ref, degc_ref,
                adj_ref, lab_ref, w1t_ref, w1b_ref, b1_ref, out_ref,
                cell_v, acc):
    i = pl.program_id(0)
    w1t = w1t_ref[...]
    w1b = w1b_ref[...]
    b1 = b1_ref[...]

    @pl.when(i == 0)
    def _():
        ce = _emb(xc_ref[...], aggc_ref[...], degc_ref[...], w1t, w1b, b1)
        cell_v[...] = ce
        m = jnp.max(ce, axis=1, keepdims=True)
        lse = m + jnp.log(jnp.sum(jnp.exp(ce - m), axis=1, keepdims=True))
        lp = ce - lse
        lab = lab_ref[...]
        io = lax.broadcasted_iota(jnp.int32, (NCELLS, H), 1)
        nll = -jnp.sum(jnp.where(io == lab, lp, 0.0), axis=1)
        smooth = -jnp.mean(lp, axis=1)
        acc[1] = jnp.mean((1.0 - LABSM) * nll + LABSM * smooth)
        acc[0] = 0.0

    gene = _emb(xg_ref[...], aggg_ref[...], degg_ref[...], w1t, w1b, b1)
    cell = cell_v[...]
    d = lax.dot_general(gene, cell, (((1,), (1,)), ((), ())),
                        preferred_element_type=jnp.float32)
    md = jnp.max(d, axis=1, keepdims=True)
    lse_d = md[:, 0] + jnp.log(jnp.sum(jnp.exp(d - md), axis=1))
    a = adj_ref[...]
    ea = jnp.exp(a)
    sa = jnp.sum(ea, axis=1)
    pa = jnp.sum(ea * a, axis=1) / sa
    u = lax.dot_general(ea, cell, (((1,), (0,)), ((), ())),
                        preferred_element_type=jnp.float32)
    pd = jnp.sum(u * gene, axis=1) / sa
    acc[0] += jnp.sum(pa - jnp.log(sa) - pd + lse_d)

    @pl.when(i == NBLK_B - 1)
    def _():
        kl = acc[0] / (jnp.float32(NCELLS) * jnp.float32(NCELLS))
        out_ref[0, 0] = 10.0 * kl + acc[1]


GOFF = NCELLS // BR_B

_tc_b2 = pl.pallas_call(
    _tc_b2_body,
    grid=(NBLK_B,),
    in_specs=[
        pl.BlockSpec((BR_B, D), lambda i: (i + GOFF, 0)),
        pl.BlockSpec((NCORE, BR_B, D), lambda i: (0, i + GOFF, 0)),
        pl.BlockSpec((NCORE, BR_B, 16), lambda i: (0, i + GOFF, 0)),
        pl.BlockSpec((NCELLS, D), lambda i: (0, 0)),
        pl.BlockSpec((NCORE, NCELLS, D), lambda i: (0, 0, 0)),
        pl.BlockSpec((NCORE, NCELLS, 16), lambda i: (0, 0, 0)),
        pl.BlockSpec((BR_B, NCELLS), lambda i: (i, 0)),
        pl.BlockSpec((NCELLS, 1), lambda i: (0, 0)),
        pl.BlockSpec((D, H), lambda i: (0, 0)),
        pl.BlockSpec((D, H), lambda i: (0, 0)),
        pl.BlockSpec((1, H), lambda i: (0, 0)),
    ],
    out_specs=pl.BlockSpec(memory_space=pltpu.SMEM),
    out_shape=jax.ShapeDtypeStruct((1, 1), jnp.float32),
    scratch_shapes=[
        pltpu.VMEM((NCELLS, H), jnp.float32),
        pltpu.SMEM((2,), jnp.float32),
    ],
)


def kernel(x, edge_index, adj_target, labels, W1, b1):
    src = jnp.asarray(edge_index[0], jnp.int32).reshape(NW, NGROUP, NSTAGE, CH)
    dst = jnp.asarray(edge_index[1], jnp.int32).reshape(NW, NGROUP, NSTAGE, CH)
    agg_parts, deg_parts = _sc_segment_sum()(x, src, dst)
    agg_parts = agg_parts.reshape(NCORE, NP, D)
    deg_parts = deg_parts.reshape(NCORE, NP, 16)
    w1t = W1[:D]
    w1b = W1[D:]
    lab = jnp.asarray(labels, jnp.int32).reshape(NCELLS, 1)
    out = _tc_b2(x, agg_parts, deg_parts, x, agg_parts, deg_parts,
                 adj_target, lab, w1t, w1b, b1.reshape(1, H))
    return out[0, 0]


# speedup vs baseline: 1.0415x; 1.0415x over previous
import functools

import jax
import jax.numpy as jnp
from jax import lax
from jax.experimental import pallas as pl
from jax.experimental.pallas import tpu as pltpu
from jax.experimental.pallas import tpu_sc as plsc

N = 10000
E = 320000
D = 128
H = 128
NCELLS = 5000
LABSM = 0.1

NCORE = 2
NSUB = 16
NW = NCORE * NSUB
EPW = E // NW
CH = 100
NSTAGE = 25
NGROUP = EPW // (CH * NSTAGE)
NCHUNK = EPW // CH
NP = 10240
ROWS_PER_SUB = NP // NSUB
ZR = 32


def _sc_body(x_hbm, src_hbm, dst_hbm, agg_out, deg_out,
             src_c, dst_c, rows0, rows1, ones_v, zb_v, zbd_v,
             agg_sp, deg_sp, sem_g, sem_s, sem_o):
    c = lax.axis_index("c")
    s = lax.axis_index("s")
    wid = s * NCORE + c

    ones16 = jnp.full((16,), 1.0, jnp.float32)
    zero16 = jnp.zeros((16,), jnp.float32)

    def fill_ones(i, carry):
        ones_v[i, :] = ones16
        return carry

    lax.fori_loop(0, CH, fill_ones, 0)

    def fill_zb(k, carry):
        zb_v[k // 8, pl.ds((k % 8) * 16, 16)] = zero16
        return carry

    lax.fori_loop(0, ZR * 8, fill_zb, 0)

    def fill_zbd(i, carry):
        zbd_v[i, :] = zero16
        return carry

    lax.fori_loop(0, ZR, fill_zbd, 0)

    def zero_slice(t, carry):
        base = s * ROWS_PER_SUB + t * ZR
        pltpu.sync_copy(zb_v, agg_sp.at[pl.ds(base, ZR)])
        pltpu.sync_copy(zbd_v, deg_sp.at[pl.ds(base, ZR)])
        return carry

    lax.fori_loop(0, ROWS_PER_SUB // ZR, zero_slice, 0)
    plsc.subcore_barrier()

    gwait = x_hbm.at[pl.ds(0, CH)]

    def group(t, carry):
        pltpu.sync_copy(src_hbm.at[wid, t], src_c)
        pltpu.sync_copy(dst_hbm.at[wid, t], dst_c)
        pltpu.async_copy(x_hbm.at[src_c.at[0]], rows0, sem_g)

        def chunk(k, carry2):
            even = k % 2 == 0
            pltpu.make_async_copy(gwait, rows0, sem_g).wait()

            @pl.when(k > 0)
            def _():
                pltpu.make_async_copy(rows0, gwait, sem_s).wait()

            @pl.when(k + 1 < NSTAGE)
            def _():
                @pl.when(even)
                def _():
                    pltpu.async_copy(x_hbm.at[src_c.at[k + 1]], rows1, sem_g)

                @pl.when(jnp.logical_not(even))
                def _():
                    pltpu.async_copy(x_hbm.at[src_c.at[k + 1]], rows0, sem_g)

            @pl.when(even)
            def _():
                pltpu.async_copy(rows0, agg_sp.at[dst_c.at[k]], sem_s,
                                 add=True)

            @pl.when(jnp.logical_not(even))
            def _():
                pltpu.async_copy(rows1, agg_sp.at[dst_c.at[k]], sem_s,
                                 add=True)

            pltpu.async_copy(ones_v, deg_sp.at[dst_c.at[k]], sem_o, add=True)
            return carry2

        lax.fori_loop(0, NSTAGE, chunk, 0)
        pltpu.make_async_copy(rows0, gwait, sem_s).wait()

        def drain(k, carry2):
            pltpu.make_async_copy(ones_v, deg_sp.at[pl.ds(0, CH)],
                                  sem_o).wait()
            return carry2

        lax.fori_loop(0, NSTAGE, drain, 0)
        return carry

    lax.fori_loop(0, NGROUP, group, 0)
    plsc.subcore_barrier()

    base = s * ROWS_PER_SUB
    pltpu.sync_copy(agg_sp.at[pl.ds(base, ROWS_PER_SUB)],
                    agg_out.at[pl.ds(c * NP + base, ROWS_PER_SUB)])
    pltpu.sync_copy(deg_sp.at[pl.ds(base, ROWS_PER_SUB)],
                    deg_out.at[pl.ds(c * NP + base, ROWS_PER_SUB)])


@functools.cache
def _sc_segment_sum():
    return pl.kernel(
        _sc_body,
        out_type=(
            jax.ShapeDtypeStruct((NCORE * NP, D), jnp.float32),
            jax.ShapeDtypeStruct((NCORE * NP, 16), jnp.float32),
        ),
        mesh=plsc.VectorSubcoreMesh(core_axis_name="c", subcore_axis_name="s"),
        compiler_params=pltpu.CompilerParams(use_tc_tiling_on_sc=False),
        scratch_types=[
            pltpu.VMEM((NSTAGE, CH), jnp.int32),
            pltpu.VMEM((NSTAGE, CH), jnp.int32),
            pltpu.VMEM((CH, D), jnp.float32),
            pltpu.VMEM((CH, D), jnp.float32),
            pltpu.VMEM((CH, 16), jnp.float32),
            pltpu.VMEM((ZR, D), jnp.float32),
            pltpu.VMEM((ZR, 16), jnp.float32),
            pltpu.VMEM_SHARED((NP, D), jnp.float32),
            pltpu.VMEM_SHARED((NP, 16), jnp.float32),
            pltpu.SemaphoreType.DMA,
            pltpu.SemaphoreType.DMA,
            pltpu.SemaphoreType.DMA,
        ],
    )


BR_B = 200
NBLK_B = NCELLS // BR_B


def _emb(xb, aggp, degp, w1t, w1b, b1):
    agg = aggp[0] + aggp[1]
    deg = degp[0, :, 0:1] + degp[1, :, 0:1]
    aggm = agg / jnp.maximum(deg, 1.0)
    h = (jnp.dot(xb, w1t, preferred_element_type=jnp.float32)
         + jnp.dot(aggm, w1b, preferred_element_type=jnp.float32)
         + b1)
    return jnp.maximum(h, 0.0)


def _tc_b2_body(xg_ref, aggg_ref, degg_ref, xc_ref, aggc_ref, degc_ref,
                adj_ref, lab_ref, w1t_ref, w1b_ref, b1_ref, out_ref,
                cell_v, acc):
    i = pl.program_id(0)
    w1t = w1t_ref[...]
    w1b = w1b_ref[...]
    b1 = b1_ref[...]

    @pl.when(i == 0)
    def _():
        ce = _emb(xc_ref[...], aggc_ref[...], degc_ref[...], w1t, w1b, b1)
        cell_v[...] = ce
        m = jnp.max(ce, axis=1, keepdims=True)
        lse = m + jnp.log(jnp.sum(jnp.exp(ce - m), axis=1, keepdims=True))
        lp = ce - lse
        lab = lab_ref[...]
        io = lax.broadcasted_iota(jnp.int32, (NCELLS, H), 1)
        nll = -jnp.sum(jnp.where(io == lab, lp, 0.0), axis=1)
        smooth = -jnp.mean(lp, axis=1)
        acc[1] = jnp.mean((1.0 - LABSM) * nll + LABSM * smooth)
        acc[0] = 0.0

    gene = _emb(xg_ref[...], aggg_ref[...], degg_ref[...], w1t, w1b, b1)
    cell = cell_v[...]
    d = lax.dot_general(gene, cell, (((1,), (1,)), ((), ())),
                        preferred_element_type=jnp.float32)
    md = jnp.max(d, axis=1, keepdims=True)
    lse_d = md[:, 0] + jnp.log(jnp.sum(jnp.exp(d - md), axis=1))
    a = adj_ref[...]
    ea = jnp.exp(a)
    sa = jnp.sum(ea, axis=1)
    pa = jnp.sum(ea * a, axis=1) / sa
    u = lax.dot_general(ea, cell, (((1,), (0,)), ((), ())),
                        preferred_element_type=jnp.float32)
    pd = jnp.sum(u * gene, axis=1) / sa
    acc[0] += jnp.sum(pa - jnp.log(sa) - pd + lse_d)

    @pl.when(i == NBLK_B - 1)
    def _():
        kl = acc[0] / (jnp.float32(NCELLS) * jnp.float32(NCELLS))
        out_ref[0, 0] = 10.0 * kl + acc[1]


GOFF = NCELLS // BR_B

_tc_b2 = pl.pallas_call(
    _tc_b2_body,
    grid=(NBLK_B,),
    in_specs=[
        pl.BlockSpec((BR_B, D), lambda i: (i + GOFF, 0)),
        pl.BlockSpec((NCORE, BR_B, D), lambda i: (0, i + GOFF, 0)),
        pl.BlockSpec((NCORE, BR_B, 16), lambda i: (0, i + GOFF, 0)),
        pl.BlockSpec((NCELLS, D), lambda i: (0, 0)),
        pl.BlockSpec((NCORE, NCELLS, D), lambda i: (0, 0, 0)),
        pl.BlockSpec((NCORE, NCELLS, 16), lambda i: (0, 0, 0)),
        pl.BlockSpec((BR_B, NCELLS), lambda i: (i, 0)),
        pl.BlockSpec((NCELLS, 1), lambda i: (0, 0)),
        pl.BlockSpec((D, H), lambda i: (0, 0)),
        pl.BlockSpec((D, H), lambda i: (0, 0)),
        pl.BlockSpec((1, H), lambda i: (0, 0)),
    ],
    out_specs=pl.BlockSpec(memory_space=pltpu.SMEM),
    out_shape=jax.ShapeDtypeStruct((1, 1), jnp.float32),
    scratch_shapes=[
        pltpu.VMEM((NCELLS, H), jnp.float32),
        pltpu.SMEM((2,), jnp.float32),
    ],
)


def kernel(x, edge_index, adj_target, labels, W1, b1):
    src = jnp.asarray(edge_index[0], jnp.int32).reshape(NW, NGROUP, NSTAGE, CH)
    dst = jnp.asarray(edge_index[1], jnp.int32).reshape(NW, NGROUP, NSTAGE, CH)
    agg_parts, deg_parts = _sc_segment_sum()(x, src, dst)
    agg_parts = agg_parts.reshape(NCORE, NP, D)
    deg_parts = deg_parts.reshape(NCORE, NP, 16)
    w1t = W1[:D]
    w1b = W1[D:]
    lab = jnp.asarray(labels, jnp.int32).reshape(NCELLS, 1)
    out = _tc_b2(x, agg_parts, deg_parts, x, agg_parts, deg_parts,
                 adj_target, lab, w1t, w1b, b1.reshape(1, H))
    return out[0, 0]
